# baseline (speedup 1.0000x reference)
import functools

import jax
import jax.numpy as jnp
from jax import lax
from jax.experimental import pallas as pl
from jax.experimental.pallas import tpu as pltpu
from jax.experimental.pallas import tpu_sc as plsc

B = 64
L = 2048
D = 128

_NC = 2
_BPC = B // _NC


def _pe_lookup(offsets, pe_flat):
    mesh = plsc.ScalarSubcoreMesh(axis_name="c", num_cores=_NC)

    @functools.partial(
        pl.kernel,
        mesh=mesh,
        out_type=jax.ShapeDtypeStruct((B * L * D,), jnp.float32),
        scratch_types=[
            pltpu.SMEM((B,), jnp.int32),
            pltpu.SemaphoreType.DMA,
        ],
    )
    def k(offs_hbm, pe_hbm, out_hbm, offs_s, sem):
        c = lax.axis_index("c")
        pltpu.sync_copy(offs_hbm, offs_s)
        copies = []
        for j in range(_BPC):
            b = c * _BPC + j
            off = offs_s[b]
            copies.append(
                pltpu.async_copy(
                    pe_hbm.at[pl.ds(off * D, L * D)],
                    out_hbm.at[pl.ds(b * (L * D), L * D)],
                    sem,
                )
            )
        for cp in copies:
            cp.wait()

    return k(offsets, pe_flat)


def kernel(x, pe):
    offsets = x[:, 0, 0].astype(jnp.int32)
    flat = _pe_lookup(offsets, pe.reshape(-1))
    return flat.reshape(B, L, D)
